# baseline (speedup 1.0000x reference)
import functools
import math

import jax
import jax.numpy as jnp
from jax import lax
from jax.experimental import pallas as pl
from jax.experimental.pallas import tpu as pltpu
from jax.experimental.pallas import tpu_sc as plsc

_SCALE = 64.0
_MARGIN = 0.5
_COS_M = math.cos(_MARGIN)
_SIN_M = math.sin(_MARGIN)
_THRESH = -math.cos(_MARGIN)
_MONO = math.sin(_MARGIN) * _MARGIN


def _tc_kernel(lab_ref, x_ref, s_ref, t_ref):
    x = x_ref[...]
    lab = lab_ref[0]
    col = lax.broadcasted_iota(jnp.int32, x.shape, 1)
    sub = col == lab
    t_ref[...] = jnp.sum(jnp.where(sub, x, 0.0), axis=1, keepdims=True)
    s_ref[...] = jnp.sum(jnp.exp(x * _SCALE - _SCALE), axis=1, keepdims=True)


def _loss_rows(s, t):
    tr = t * _COS_M - _SIN_M * jnp.sqrt(jnp.maximum(1.0 - t * t, 0.0))
    tr = jnp.where(t > _THRESH, tr, t - _MONO)
    s = s - jnp.exp(t * _SCALE - _SCALE) + jnp.exp(tr * _SCALE - _SCALE)
    return jnp.log(s) + _SCALE - _SCALE * tr


def _combine_kernel(stc_ref, ttc_ref, sc_ref, tail_ref, lab_ref, out_ref, *,
                    B, B_sc, C_sc, C):
    l_tc = _loss_rows(stc_ref[...], ttc_ref[...])
    tail = tail_ref[...]
    lab2 = lab_ref[...]
    col = lax.broadcasted_iota(jnp.int32, tail.shape, 1) + C_sc
    s_tail = jnp.sum(jnp.where(col < C, jnp.exp(tail * _SCALE - _SCALE), 0.0),
                     axis=1, keepdims=True)
    t_tail = jnp.sum(jnp.where(col == lab2, tail, 0.0), axis=1, keepdims=True)
    sc = sc_ref[...]
    s = jnp.sum(sc[:B_sc, :], axis=1, keepdims=True) + s_tail
    t = jnp.sum(sc[B_sc:, :], axis=1, keepdims=True) + t_tail
    l_sc = _loss_rows(s, t)
    out_ref[...] = (jnp.sum(l_tc) + jnp.sum(l_sc)).reshape(1, 1) / B


def _sc_dense(cosine, labrep, B_tc, B_sc, C_sc):
    info = plsc.get_sparse_core_info()
    NC = info.num_cores
    NW = NC * info.num_subcores
    bpw = B_sc // NW
    CH = 4992
    chunks = [(k * CH, CH) for k in range(C_sc // CH)]
    if C_sc % CH:
        chunks.append((C_sc - C_sc % CH, C_sc % CH))
    n_ch = len(chunks)
    mesh = plsc.VectorSubcoreMesh(core_axis_name="c", subcore_axis_name="s")

    @functools.partial(
        pl.kernel,
        mesh=mesh,
        out_type=jax.ShapeDtypeStruct((2 * B_sc * 16,), jnp.float32),
        scratch_types=[
            pltpu.VMEM((bpw, 16), jnp.float32),
            pltpu.VMEM((bpw, 16), jnp.float32),
            pltpu.VMEM((bpw, CH), jnp.float32),
            pltpu.VMEM((bpw, CH), jnp.float32),
            pltpu.VMEM((bpw * 16,), jnp.float32),
            pltpu.VMEM((bpw * 16,), jnp.float32),
            pltpu.SemaphoreType.DMA,
            pltpu.SemaphoreType.DMA,
        ],
    )
    def sc_k(cos_hbm, seg_hbm, oh_hbm, out_hbm, segb, ohb, buf0, buf1,
             svb, tvb, sem0, sem1):
        wid = lax.axis_index("s") * NC + lax.axis_index("c")
        base = wid * bpw
        row0 = B_tc + base
        pltpu.sync_copy(seg_hbm.at[pl.ds(base, bpw)], segb)
        pltpu.sync_copy(oh_hbm.at[pl.ds(base, bpw)], ohb)

        bufs = [buf0, buf1]
        sems = [sem0, sem1]

        def src(g):
            start, width = chunks[g]
            return cos_hbm.at[pl.ds(row0, bpw), pl.ds(start, width)]

        def dst(g):
            width = chunks[g][1]
            return bufs[g % 2].at[pl.ds(0, bpw), pl.ds(0, width)]

        accs = tuple(jnp.zeros((16,), jnp.float32) for _ in range(2 * bpw))
        cps = {0: pltpu.async_copy(src(0), dst(0), sems[0])}
        for g in range(n_ch):
            start, width = chunks[g]
            if g + 1 < n_ch:
                cps[g + 1] = pltpu.async_copy(src(g + 1), dst(g + 1),
                                              sems[(g + 1) % 2])
            cps[g].wait()
            buf = bufs[g % 2]
            chb = start // 16

            def body(i, a, _buf=buf):
                out = []
                tout = []
                ivec = a[2 * bpw]
                for r in range(bpw):
                    v = _buf[r, pl.ds(i * 16, 16)]
                    out.append(a[r] + jnp.exp(v * _SCALE - _SCALE))
                    m = jnp.maximum(
                        1.0 - jnp.abs(segb[r, pl.ds(0, 16)] - ivec), 0.0)
                    tout.append(a[bpw + r] + v * (ohb[r, pl.ds(0, 16)] * m))
                return tuple(out + tout + [ivec + 1.0])

            res = lax.fori_loop(0, width // 16, body,
                                accs + (jnp.full((16,), float(chb),
                                                 jnp.float32),))
            accs = res[:2 * bpw]

        for j in range(bpw):
            svb[pl.ds(j * 16, 16)] = accs[j]
            tvb[pl.ds(j * 16, 16)] = accs[bpw + j]
        pltpu.sync_copy(svb, out_hbm.at[pl.ds(base * 16, bpw * 16)])
        pltpu.sync_copy(tvb,
                        out_hbm.at[pl.ds((B_sc + base) * 16, bpw * 16)])

    return sc_k(cosine, labrep[0], labrep[1])


@jax.jit
def kernel(cosine, label):
    B, C = cosine.shape
    B_sc = 256
    B_tc = B - B_sc
    BR = 16
    R = B_tc // BR

    C_sc = (C // 128) * 128
    label = label.astype(jnp.int32)
    lsc = label[B_tc:]
    segrep = jnp.broadcast_to((lsc // 16).astype(jnp.float32)[:, None],
                              (B_sc, 16))
    ohrep = ((lsc % 16)[:, None] == jnp.arange(16)[None, :]).astype(jnp.float32)
    lab3 = label[:B_tc].reshape(R, BR, 1)
    s_tc, t_tc = pl.pallas_call(
        _tc_kernel,
        grid=(R,),
        in_specs=[
            pl.BlockSpec((1, BR, 1), lambda r: (r, 0, 0)),
            pl.BlockSpec((BR, C), lambda r: (r, 0)),
        ],
        out_specs=[
            pl.BlockSpec((BR, 1), lambda r: (r, 0)),
            pl.BlockSpec((BR, 1), lambda r: (r, 0)),
        ],
        out_shape=[
            jax.ShapeDtypeStruct((B_tc, 1), jnp.float32),
            jax.ShapeDtypeStruct((B_tc, 1), jnp.float32),
        ],
    )(lab3, cosine)

    sc_out = _sc_dense(cosine, (segrep, ohrep), B_tc, B_sc,
                       C_sc).reshape(2 * B_sc, 16)

    lab_sc = label[B_tc:].reshape(B_sc, 1)
    out = pl.pallas_call(
        functools.partial(_combine_kernel, B=B, B_sc=B_sc, C_sc=C_sc, C=C),
        grid=(1,),
        in_specs=[
            pl.BlockSpec((B_tc, 1), lambda i: (0, 0)),
            pl.BlockSpec((B_tc, 1), lambda i: (0, 0)),
            pl.BlockSpec((2 * B_sc, 16), lambda i: (0, 0)),
            pl.BlockSpec((B_sc, 128), lambda i: (B_tc // B_sc, C_sc // 128)),
            pl.BlockSpec((B_sc, 1), lambda i: (0, 0)),
        ],
        out_specs=pl.BlockSpec((1, 1), lambda i: (0, 0)),
        out_shape=jax.ShapeDtypeStruct((1, 1), jnp.float32),
    )(s_tc, t_tc, sc_out, cosine, lab_sc)
    return out[0, 0]


# speedup vs baseline: 1.0271x; 1.0271x over previous
import functools
import math

import jax
import jax.numpy as jnp
from jax import lax
from jax.experimental import pallas as pl
from jax.experimental.pallas import tpu as pltpu
from jax.experimental.pallas import tpu_sc as plsc

_SCALE = 64.0
_MARGIN = 0.5
_COS_M = math.cos(_MARGIN)
_SIN_M = math.sin(_MARGIN)
_THRESH = -math.cos(_MARGIN)
_MONO = math.sin(_MARGIN) * _MARGIN


def _tc_kernel(lab_ref, x_ref, s_ref, t_ref):
    x = x_ref[...]
    lab = lab_ref[0]
    col = lax.broadcasted_iota(jnp.int32, x.shape, 1)
    sub = col == lab
    t_ref[...] = jnp.sum(jnp.where(sub, x, 0.0), axis=1, keepdims=True)
    s_ref[...] = jnp.sum(jnp.exp(x * _SCALE - _SCALE), axis=1, keepdims=True)


def _loss_rows(s, t):
    tr = t * _COS_M - _SIN_M * jnp.sqrt(jnp.maximum(1.0 - t * t, 0.0))
    tr = jnp.where(t > _THRESH, tr, t - _MONO)
    s = s - jnp.exp(t * _SCALE - _SCALE) + jnp.exp(tr * _SCALE - _SCALE)
    return jnp.log(s) + _SCALE - _SCALE * tr


def _combine_kernel(stc_ref, ttc_ref, sc_ref, tail_ref, lab_ref, out_ref, *,
                    B, B_sc, C_sc, C):
    l_tc = _loss_rows(stc_ref[...], ttc_ref[...])
    tail = tail_ref[...]
    lab2 = lab_ref[...]
    col = lax.broadcasted_iota(jnp.int32, tail.shape, 1) + C_sc
    s_tail = jnp.sum(jnp.where(col < C, jnp.exp(tail * _SCALE - _SCALE), 0.0),
                     axis=1, keepdims=True)
    t_tail = jnp.sum(jnp.where(col == lab2, tail, 0.0), axis=1, keepdims=True)
    sc = sc_ref[...]
    s = jnp.sum(sc[:B_sc, :], axis=1, keepdims=True) + s_tail
    t = jnp.sum(sc[B_sc:, :], axis=1, keepdims=True) + t_tail
    l_sc = _loss_rows(s, t)
    out_ref[...] = (jnp.sum(l_tc) + jnp.sum(l_sc)).reshape(1, 1) / B


def _sc_dense(cosine, labrep, B_tc, B_sc, C_sc):
    info = plsc.get_sparse_core_info()
    NC = info.num_cores
    NW = NC * info.num_subcores
    bpw = B_sc // NW
    CH = 4992
    chunks = [(k * CH, CH) for k in range(C_sc // CH)]
    if C_sc % CH:
        chunks.append((C_sc - C_sc % CH, C_sc % CH))
    n_ch = len(chunks)
    mesh = plsc.VectorSubcoreMesh(core_axis_name="c", subcore_axis_name="s")

    @functools.partial(
        pl.kernel,
        mesh=mesh,
        out_type=jax.ShapeDtypeStruct((2 * B_sc * 16,), jnp.float32),
        scratch_types=[
            pltpu.VMEM((bpw, 16), jnp.float32),
            pltpu.VMEM((bpw, 16), jnp.float32),
            pltpu.VMEM((bpw, CH), jnp.float32),
            pltpu.VMEM((bpw, CH), jnp.float32),
            pltpu.VMEM((bpw * 16,), jnp.float32),
            pltpu.VMEM((bpw * 16,), jnp.float32),
            pltpu.SemaphoreType.DMA,
            pltpu.SemaphoreType.DMA,
        ],
    )
    def sc_k(cos_hbm, seg_hbm, oh_hbm, out_hbm, segb, ohb, buf0, buf1,
             svb, tvb, sem0, sem1):
        wid = lax.axis_index("s") * NC + lax.axis_index("c")
        base = wid * bpw
        row0 = B_tc + base
        pltpu.sync_copy(seg_hbm.at[pl.ds(base, bpw)], segb)
        pltpu.sync_copy(oh_hbm.at[pl.ds(base, bpw)], ohb)

        bufs = [buf0, buf1]
        sems = [sem0, sem1]

        def src(g):
            start, width = chunks[g]
            return cos_hbm.at[pl.ds(row0, bpw), pl.ds(start, width)]

        def dst(g):
            width = chunks[g][1]
            return bufs[g % 2].at[pl.ds(0, bpw), pl.ds(0, width)]

        accs = tuple(jnp.zeros((16,), jnp.float32) for _ in range(2 * bpw))
        cps = {0: pltpu.async_copy(src(0), dst(0), sems[0])}
        for g in range(n_ch):
            start, width = chunks[g]
            if g + 1 < n_ch:
                cps[g + 1] = pltpu.async_copy(src(g + 1), dst(g + 1),
                                              sems[(g + 1) % 2])
            cps[g].wait()
            buf = bufs[g % 2]
            chb = start // 16

            def body(i, a, _buf=buf):
                out = []
                tout = []
                ivec = a[2 * bpw]
                for r in range(bpw):
                    v = _buf[r, pl.ds(i * 16, 16)]
                    out.append(a[r] + jnp.exp(v * _SCALE - _SCALE))
                    m = jnp.maximum(
                        1.0 - jnp.abs(segb[r, pl.ds(0, 16)] - ivec), 0.0)
                    tout.append(a[bpw + r] + v * (ohb[r, pl.ds(0, 16)] * m))
                return tuple(out + tout + [ivec + 1.0])

            res = lax.fori_loop(0, width // 16, body,
                                accs + (jnp.full((16,), float(chb),
                                                 jnp.float32),))
            accs = res[:2 * bpw]

        for j in range(bpw):
            svb[pl.ds(j * 16, 16)] = accs[j]
            tvb[pl.ds(j * 16, 16)] = accs[bpw + j]
        pltpu.sync_copy(svb, out_hbm.at[pl.ds(base * 16, bpw * 16)])
        pltpu.sync_copy(tvb,
                        out_hbm.at[pl.ds((B_sc + base) * 16, bpw * 16)])

    return sc_k(cosine, labrep[0], labrep[1])


@jax.jit
def kernel(cosine, label):
    B, C = cosine.shape
    B_sc = 256
    B_tc = B - B_sc
    BR = 32
    R = B_tc // BR

    C_sc = (C // 128) * 128
    label = label.astype(jnp.int32)
    lsc = label[B_tc:]
    segrep = jnp.broadcast_to((lsc // 16).astype(jnp.float32)[:, None],
                              (B_sc, 16))
    ohrep = ((lsc % 16)[:, None] == jnp.arange(16)[None, :]).astype(jnp.float32)
    lab3 = label[:B_tc].reshape(R, BR, 1)
    s_tc, t_tc = pl.pallas_call(
        _tc_kernel,
        grid=(R,),
        in_specs=[
            pl.BlockSpec((1, BR, 1), lambda r: (r, 0, 0)),
            pl.BlockSpec((BR, C), lambda r: (r, 0)),
        ],
        out_specs=[
            pl.BlockSpec((BR, 1), lambda r: (r, 0)),
            pl.BlockSpec((BR, 1), lambda r: (r, 0)),
        ],
        out_shape=[
            jax.ShapeDtypeStruct((B_tc, 1), jnp.float32),
            jax.ShapeDtypeStruct((B_tc, 1), jnp.float32),
        ],
    )(lab3, cosine)

    sc_out = _sc_dense(cosine, (segrep, ohrep), B_tc, B_sc,
                       C_sc).reshape(2 * B_sc, 16)

    lab_sc = label[B_tc:].reshape(B_sc, 1)
    out = pl.pallas_call(
        functools.partial(_combine_kernel, B=B, B_sc=B_sc, C_sc=C_sc, C=C),
        grid=(1,),
        in_specs=[
            pl.BlockSpec((B_tc, 1), lambda i: (0, 0)),
            pl.BlockSpec((B_tc, 1), lambda i: (0, 0)),
            pl.BlockSpec((2 * B_sc, 16), lambda i: (0, 0)),
            pl.BlockSpec((B_sc, 128), lambda i: (B_tc // B_sc, C_sc // 128)),
            pl.BlockSpec((B_sc, 1), lambda i: (0, 0)),
        ],
        out_specs=pl.BlockSpec((1, 1), lambda i: (0, 0)),
        out_shape=jax.ShapeDtypeStruct((1, 1), jnp.float32),
    )(s_tc, t_tc, sc_out, cosine, lab_sc)
    return out[0, 0]


# speedup vs baseline: 1.0278x; 1.0007x over previous
import functools
import math

import jax
import jax.numpy as jnp
from jax import lax
from jax.experimental import pallas as pl
from jax.experimental.pallas import tpu as pltpu
from jax.experimental.pallas import tpu_sc as plsc

_SCALE = 64.0
_MARGIN = 0.5
_COS_M = math.cos(_MARGIN)
_SIN_M = math.sin(_MARGIN)
_THRESH = -math.cos(_MARGIN)
_MONO = math.sin(_MARGIN) * _MARGIN


def _tc_kernel(lab_ref, x_ref, s_ref, t_ref):
    x = x_ref[...]
    lab = lab_ref[0]
    col = lax.broadcasted_iota(jnp.int32, x.shape, 1)
    sub = col == lab
    t_ref[...] = jnp.sum(jnp.where(sub, x, 0.0), axis=1, keepdims=True)
    s_ref[...] = jnp.sum(jnp.exp(x * _SCALE - _SCALE), axis=1, keepdims=True)


def _loss_rows(s, t):
    tr = t * _COS_M - _SIN_M * jnp.sqrt(jnp.maximum(1.0 - t * t, 0.0))
    tr = jnp.where(t > _THRESH, tr, t - _MONO)
    s = s - jnp.exp(t * _SCALE - _SCALE) + jnp.exp(tr * _SCALE - _SCALE)
    return jnp.log(s) + _SCALE - _SCALE * tr


def _combine_kernel(stc_ref, ttc_ref, sc_ref, tail_ref, lab_ref, out_ref, *,
                    B, B_sc, C_sc, C):
    l_tc = _loss_rows(stc_ref[...], ttc_ref[...])
    tail = tail_ref[...]
    lab2 = lab_ref[...]
    col = lax.broadcasted_iota(jnp.int32, tail.shape, 1) + C_sc
    s_tail = jnp.sum(jnp.where(col < C, jnp.exp(tail * _SCALE - _SCALE), 0.0),
                     axis=1, keepdims=True)
    t_tail = jnp.sum(jnp.where(col == lab2, tail, 0.0), axis=1, keepdims=True)
    sc = sc_ref[...]
    s = jnp.sum(sc[:B_sc, :], axis=1, keepdims=True) + s_tail
    t = jnp.sum(sc[B_sc:, :], axis=1, keepdims=True) + t_tail
    l_sc = _loss_rows(s, t)
    out_ref[...] = (jnp.sum(l_tc) + jnp.sum(l_sc)).reshape(1, 1) / B


def _sc_dense(cosine, labrep, B_tc, B_sc, C_sc):
    info = plsc.get_sparse_core_info()
    NC = info.num_cores
    NW = NC * info.num_subcores
    bpw = B_sc // NW
    CH = 4992
    chunks = [(k * CH, CH) for k in range(C_sc // CH)]
    if C_sc % CH:
        chunks.append((C_sc - C_sc % CH, C_sc % CH))
    n_ch = len(chunks)
    mesh = plsc.VectorSubcoreMesh(core_axis_name="c", subcore_axis_name="s")

    @functools.partial(
        pl.kernel,
        mesh=mesh,
        out_type=jax.ShapeDtypeStruct((2 * B_sc * 16,), jnp.float32),
        scratch_types=[
            pltpu.VMEM((bpw, 16), jnp.float32),
            pltpu.VMEM((bpw, 16), jnp.float32),
            pltpu.VMEM((bpw, CH), jnp.float32),
            pltpu.VMEM((bpw, CH), jnp.float32),
            pltpu.VMEM((bpw * 16,), jnp.float32),
            pltpu.VMEM((bpw * 16,), jnp.float32),
            pltpu.SemaphoreType.DMA,
            pltpu.SemaphoreType.DMA,
        ],
    )
    def sc_k(cos_hbm, seg_hbm, oh_hbm, out_hbm, segb, ohb, buf0, buf1,
             svb, tvb, sem0, sem1):
        wid = lax.axis_index("s") * NC + lax.axis_index("c")
        base = wid * bpw
        row0 = B_tc + base
        pltpu.sync_copy(seg_hbm.at[pl.ds(base, bpw)], segb)
        pltpu.sync_copy(oh_hbm.at[pl.ds(base, bpw)], ohb)

        bufs = [buf0, buf1]
        sems = [sem0, sem1]

        def src(g):
            start, width = chunks[g]
            return cos_hbm.at[pl.ds(row0, bpw), pl.ds(start, width)]

        def dst(g):
            width = chunks[g][1]
            return bufs[g % 2].at[pl.ds(0, bpw), pl.ds(0, width)]

        accs = tuple(jnp.zeros((16,), jnp.float32) for _ in range(2 * bpw))
        cps = {0: pltpu.async_copy(src(0), dst(0), sems[0])}
        for g in range(n_ch):
            start, width = chunks[g]
            if g + 1 < n_ch:
                cps[g + 1] = pltpu.async_copy(src(g + 1), dst(g + 1),
                                              sems[(g + 1) % 2])
            cps[g].wait()
            buf = bufs[g % 2]
            chb = start // 16

            def body(i, a, _buf=buf):
                out = []
                tout = []
                ivec = a[2 * bpw]
                for r in range(bpw):
                    v = _buf[r, pl.ds(i * 16, 16)]
                    out.append(a[r] + jnp.exp(v * _SCALE - _SCALE))
                    m = jnp.maximum(
                        1.0 - jnp.abs(segb[r, pl.ds(0, 16)] - ivec), 0.0)
                    tout.append(a[bpw + r] + v * (ohb[r, pl.ds(0, 16)] * m))
                return tuple(out + tout + [ivec + 1.0])

            res = lax.fori_loop(0, width // 16, body,
                                accs + (jnp.full((16,), float(chb),
                                                 jnp.float32),))
            accs = res[:2 * bpw]

        for j in range(bpw):
            svb[pl.ds(j * 16, 16)] = accs[j]
            tvb[pl.ds(j * 16, 16)] = accs[bpw + j]
        pltpu.sync_copy(svb, out_hbm.at[pl.ds(base * 16, bpw * 16)])
        pltpu.sync_copy(tvb,
                        out_hbm.at[pl.ds((B_sc + base) * 16, bpw * 16)])

    return sc_k(cosine, labrep[0], labrep[1])


@jax.jit
def kernel(cosine, label):
    B, C = cosine.shape
    B_sc = 256
    B_tc = B - B_sc
    BR = 64
    R = B_tc // BR

    C_sc = (C // 128) * 128
    label = label.astype(jnp.int32)
    lsc = label[B_tc:]
    segrep = jnp.broadcast_to((lsc // 16).astype(jnp.float32)[:, None],
                              (B_sc, 16))
    ohrep = ((lsc % 16)[:, None] == jnp.arange(16)[None, :]).astype(jnp.float32)
    lab3 = label[:B_tc].reshape(R, BR, 1)
    s_tc, t_tc = pl.pallas_call(
        _tc_kernel,
        grid=(R,),
        in_specs=[
            pl.BlockSpec((1, BR, 1), lambda r: (r, 0, 0)),
            pl.BlockSpec((BR, C), lambda r: (r, 0)),
        ],
        out_specs=[
            pl.BlockSpec((BR, 1), lambda r: (r, 0)),
            pl.BlockSpec((BR, 1), lambda r: (r, 0)),
        ],
        out_shape=[
            jax.ShapeDtypeStruct((B_tc, 1), jnp.float32),
            jax.ShapeDtypeStruct((B_tc, 1), jnp.float32),
        ],
    )(lab3, cosine)

    sc_out = _sc_dense(cosine, (segrep, ohrep), B_tc, B_sc,
                       C_sc).reshape(2 * B_sc, 16)

    lab_sc = label[B_tc:].reshape(B_sc, 1)
    out = pl.pallas_call(
        functools.partial(_combine_kernel, B=B, B_sc=B_sc, C_sc=C_sc, C=C),
        grid=(1,),
        in_specs=[
            pl.BlockSpec((B_tc, 1), lambda i: (0, 0)),
            pl.BlockSpec((B_tc, 1), lambda i: (0, 0)),
            pl.BlockSpec((2 * B_sc, 16), lambda i: (0, 0)),
            pl.BlockSpec((B_sc, 128), lambda i: (B_tc // B_sc, C_sc // 128)),
            pl.BlockSpec((B_sc, 1), lambda i: (0, 0)),
        ],
        out_specs=pl.BlockSpec((1, 1), lambda i: (0, 0)),
        out_shape=jax.ShapeDtypeStruct((1, 1), jnp.float32),
    )(s_tc, t_tc, sc_out, cosine, lab_sc)
    return out[0, 0]


# speedup vs baseline: 1.0288x; 1.0010x over previous
import functools
import math

import jax
import jax.numpy as jnp
from jax import lax
from jax.experimental import pallas as pl
from jax.experimental.pallas import tpu as pltpu
from jax.experimental.pallas import tpu_sc as plsc

_SCALE = 64.0
_MARGIN = 0.5
_COS_M = math.cos(_MARGIN)
_SIN_M = math.sin(_MARGIN)
_THRESH = -math.cos(_MARGIN)
_MONO = math.sin(_MARGIN) * _MARGIN


def _tc_kernel(lab_ref, x_ref, s_ref, t_ref):
    x = x_ref[...]
    lab = lab_ref[0]
    col = lax.broadcasted_iota(jnp.int32, x.shape, 1)
    sub = col == lab
    t_ref[...] = jnp.sum(jnp.where(sub, x, 0.0), axis=1, keepdims=True)
    s_ref[...] = jnp.sum(jnp.exp(x * _SCALE - _SCALE), axis=1, keepdims=True)


def _loss_rows(s, t):
    tr = t * _COS_M - _SIN_M * jnp.sqrt(jnp.maximum(1.0 - t * t, 0.0))
    tr = jnp.where(t > _THRESH, tr, t - _MONO)
    s = s - jnp.exp(t * _SCALE - _SCALE) + jnp.exp(tr * _SCALE - _SCALE)
    return jnp.log(s) + _SCALE - _SCALE * tr


def _combine_kernel(stc_ref, ttc_ref, sc_ref, tail_ref, lab_ref, out_ref, *,
                    B, B_sc, C_sc, C):
    l_tc = _loss_rows(stc_ref[...], ttc_ref[...])
    tail = tail_ref[...]
    lab2 = lab_ref[...]
    col = lax.broadcasted_iota(jnp.int32, tail.shape, 1) + C_sc
    s_tail = jnp.sum(jnp.where(col < C, jnp.exp(tail * _SCALE - _SCALE), 0.0),
                     axis=1, keepdims=True)
    t_tail = jnp.sum(jnp.where(col == lab2, tail, 0.0), axis=1, keepdims=True)
    sc = sc_ref[...]
    s = jnp.sum(sc[:B_sc, :], axis=1, keepdims=True) + s_tail
    t = jnp.sum(sc[B_sc:, :], axis=1, keepdims=True) + t_tail
    l_sc = _loss_rows(s, t)
    out_ref[...] = (jnp.sum(l_tc) + jnp.sum(l_sc)).reshape(1, 1) / B


def _sc_dense(cosine, labrep, B_tc, B_sc, C_sc):
    info = plsc.get_sparse_core_info()
    NC = info.num_cores
    NW = NC * info.num_subcores
    bpw = B_sc // NW
    CH = 4992
    chunks = [(k * CH, CH) for k in range(C_sc // CH)]
    if C_sc % CH:
        chunks.append((C_sc - C_sc % CH, C_sc % CH))
    n_ch = len(chunks)
    mesh = plsc.VectorSubcoreMesh(core_axis_name="c", subcore_axis_name="s")

    @functools.partial(
        pl.kernel,
        mesh=mesh,
        out_type=jax.ShapeDtypeStruct((2 * B_sc * 16,), jnp.float32),
        scratch_types=[
            pltpu.VMEM((bpw, 16), jnp.float32),
            pltpu.VMEM((bpw, 16), jnp.float32),
            pltpu.VMEM((bpw, CH), jnp.float32),
            pltpu.VMEM((bpw, CH), jnp.float32),
            pltpu.VMEM((bpw * 16,), jnp.float32),
            pltpu.VMEM((bpw * 16,), jnp.float32),
            pltpu.SemaphoreType.DMA,
            pltpu.SemaphoreType.DMA,
        ],
    )
    def sc_k(cos_hbm, seg_hbm, oh_hbm, out_hbm, segb, ohb, buf0, buf1,
             svb, tvb, sem0, sem1):
        wid = lax.axis_index("s") * NC + lax.axis_index("c")
        base = wid * bpw
        row0 = B_tc + base
        pltpu.sync_copy(seg_hbm.at[pl.ds(base, bpw)], segb)
        pltpu.sync_copy(oh_hbm.at[pl.ds(base, bpw)], ohb)

        bufs = [buf0, buf1]
        sems = [sem0, sem1]

        def src(g):
            start, width = chunks[g]
            return cos_hbm.at[pl.ds(row0, bpw), pl.ds(start, width)]

        def dst(g):
            width = chunks[g][1]
            return bufs[g % 2].at[pl.ds(0, bpw), pl.ds(0, width)]

        accs = tuple(jnp.zeros((16,), jnp.float32) for _ in range(2 * bpw))
        cps = {0: pltpu.async_copy(src(0), dst(0), sems[0])}
        for g in range(n_ch):
            start, width = chunks[g]
            if g + 1 < n_ch:
                cps[g + 1] = pltpu.async_copy(src(g + 1), dst(g + 1),
                                              sems[(g + 1) % 2])
            cps[g].wait()
            buf = bufs[g % 2]
            chb = start // 16

            def body(i, a, _buf=buf):
                out = []
                tout = []
                ivec = a[2 * bpw]
                for r in range(bpw):
                    v = _buf[r, pl.ds(i * 16, 16)]
                    out.append(a[r] + jnp.exp(v * _SCALE - _SCALE))
                    m = jnp.maximum(
                        1.0 - jnp.abs(segb[r, pl.ds(0, 16)] - ivec), 0.0)
                    tout.append(a[bpw + r] + v * (ohb[r, pl.ds(0, 16)] * m))
                return tuple(out + tout + [ivec + 1.0])

            res = lax.fori_loop(0, width // 16, body,
                                accs + (jnp.full((16,), float(chb),
                                                 jnp.float32),))
            accs = res[:2 * bpw]

        for j in range(bpw):
            svb[pl.ds(j * 16, 16)] = accs[j]
            tvb[pl.ds(j * 16, 16)] = accs[bpw + j]
        pltpu.sync_copy(svb, out_hbm.at[pl.ds(base * 16, bpw * 16)])
        pltpu.sync_copy(tvb,
                        out_hbm.at[pl.ds((B_sc + base) * 16, bpw * 16)])

    return sc_k(cosine, labrep[0], labrep[1])


@jax.jit
def kernel(cosine, label):
    B, C = cosine.shape
    B_sc = 256
    B_tc = B - B_sc
    BR = 32
    R = B_tc // BR

    C_sc = (C // 128) * 128
    label = label.astype(jnp.int32)
    lsc = label[B_tc:]
    segrep = jnp.broadcast_to((lsc // 16).astype(jnp.float32)[:, None],
                              (B_sc, 16))
    ohrep = ((lsc % 16)[:, None] == jnp.arange(16)[None, :]).astype(jnp.float32)
    lab3 = label[:B_tc].reshape(R, BR, 1)
    s_tc, t_tc = pl.pallas_call(
        _tc_kernel,
        grid=(R,),
        in_specs=[
            pl.BlockSpec((1, BR, 1), lambda r: (r, 0, 0)),
            pl.BlockSpec((BR, C), lambda r: (r, 0)),
        ],
        out_specs=[
            pl.BlockSpec((BR, 1), lambda r: (r, 0)),
            pl.BlockSpec((BR, 1), lambda r: (r, 0)),
        ],
        out_shape=[
            jax.ShapeDtypeStruct((B_tc, 1), jnp.float32),
            jax.ShapeDtypeStruct((B_tc, 1), jnp.float32),
        ],
    )(lab3, cosine)

    sc_out = _sc_dense(cosine, (segrep, ohrep), B_tc, B_sc,
                       C_sc).reshape(2 * B_sc, 16)

    lab_sc = label[B_tc:].reshape(B_sc, 1)
    out = pl.pallas_call(
        functools.partial(_combine_kernel, B=B, B_sc=B_sc, C_sc=C_sc, C=C),
        grid=(1,),
        in_specs=[
            pl.BlockSpec((B_tc, 1), lambda i: (0, 0)),
            pl.BlockSpec((B_tc, 1), lambda i: (0, 0)),
            pl.BlockSpec((2 * B_sc, 16), lambda i: (0, 0)),
            pl.BlockSpec((B_sc, 128), lambda i: (B_tc // B_sc, C_sc // 128)),
            pl.BlockSpec((B_sc, 1), lambda i: (0, 0)),
        ],
        out_specs=pl.BlockSpec((1, 1), lambda i: (0, 0)),
        out_shape=jax.ShapeDtypeStruct((1, 1), jnp.float32),
    )(s_tc, t_tc, sc_out, cosine, lab_sc)
    return out[0, 0]
